# baseline (speedup 1.0000x reference)
import jax
import jax.numpy as jnp
from jax.experimental import pallas as pl
from jax.experimental.pallas import tpu as pltpu

_BLK = 4096
_NB = 3


def _plda_manual(s_ref, x_hbm, u_ref, y_hbm, z_hbm, xb, yb, zb, si, sy, sz):
    nblk = x_hbm.shape[0] // _BLK
    s = s_ref[0]
    u = u_ref[...]

    def load(i, j):
        return [
            pltpu.make_async_copy(
                x_hbm.at[pl.ds(i * _BLK, _BLK)], xb.at[j], si.at[j]
            )
        ]

    def store_y(i, j):
        return [
            pltpu.make_async_copy(
                yb.at[j], y_hbm.at[pl.ds(i * _BLK, _BLK)], sy.at[j]
            )
        ]

    def store_z(i, j):
        return [
            pltpu.make_async_copy(
                zb.at[j], z_hbm.at[pl.ds(i * _BLK, _BLK)], sz.at[j]
            )
        ]

    def start(descs):
        for d in descs:
            d.start()

    def wait(descs):
        for d in descs:
            d.wait()

    for j in range(_NB):
        start(load(j, j))

    for i in range(nblk):
        j = i % _NB
        wait(load(i, j))
        if i >= _NB:
            wait(store_y(i - _NB, j))
            wait(store_z(i - _NB, j))
        x = xb[j]
        norm = jnp.sqrt(jnp.sum(x * x, axis=1, keepdims=True))
        norm = jnp.maximum(norm, 1e-12)
        y = (s / norm) * x
        yb[j] = y
        start(store_y(i, j))
        zb[j] = jnp.dot(y, u, preferred_element_type=jnp.float32)
        start(store_z(i, j))
        if i + _NB < nblk:
            start(load(i + _NB, j))

    for i in range(max(nblk - _NB, 0), nblk):
        j = i % _NB
        wait(store_y(i, j))
        wait(store_z(i, j))


def kernel(x, norm_scale, Ulda):
    batch, dim = x.shape
    scale = jnp.reshape(norm_scale.astype(jnp.float32), (1,))
    y, z = pl.pallas_call(
        _plda_manual,
        in_specs=[
            pl.BlockSpec(memory_space=pltpu.SMEM),
            pl.BlockSpec(memory_space=pl.ANY),
            pl.BlockSpec(memory_space=pltpu.VMEM),
        ],
        out_specs=[
            pl.BlockSpec(memory_space=pl.ANY),
            pl.BlockSpec(memory_space=pl.ANY),
        ],
        out_shape=[
            jax.ShapeDtypeStruct((batch, dim), jnp.float32),
            jax.ShapeDtypeStruct((batch, dim), jnp.float32),
        ],
        scratch_shapes=[
            pltpu.VMEM((_NB, _BLK, dim), jnp.float32),
            pltpu.VMEM((_NB, _BLK, dim), jnp.float32),
            pltpu.VMEM((_NB, _BLK, dim), jnp.float32),
            pltpu.SemaphoreType.DMA((_NB,)),
            pltpu.SemaphoreType.DMA((_NB,)),
            pltpu.SemaphoreType.DMA((_NB,)),
        ],
    )(scale, x, Ulda)
    return (y, z)
